# baseline (speedup 1.0000x reference)
import functools

import jax
import jax.numpy as jnp
from jax import lax
from jax.experimental import pallas as pl
from jax.experimental.pallas import tpu as pltpu
from jax.experimental.pallas import tpu_sc as plsc

_N = 10000
_DEG = 32
_D = 128
_BN = 400

_NC = 2
_NS = 16
_NW = _NC * _NS
_A = 3200
_PER_W = _A // _NW
_CHUNK = 10
_NCHUNKS = _PER_W // _CHUNK


def _sc_mean_body(neigh_hbm, out_hbm, buf0, buf1, obuf, sem0, sem1):
    wid = lax.axis_index("s") * _NC + lax.axis_index("c")
    gbase = wid * _PER_W

    def start(c, buf, sem):
        pltpu.async_copy(neigh_hbm.at[pl.ds(gbase + c * _CHUNK, _CHUNK)], buf, sem)

    def wait(buf, sem):
        pltpu.make_async_copy(
            neigh_hbm.at[pl.ds(gbase, _CHUNK)], buf, sem
        ).wait()

    def compute(c, buf):
        for n in range(_CHUNK):
            def rbody(r, accs):
                return tuple(
                    accs[g] + buf[n, r, pl.ds(g * 16, 16)] for g in range(8)
                )
            accs = lax.fori_loop(
                0, _DEG, rbody,
                tuple(jnp.zeros((16,), jnp.float32) for _ in range(8)),
            )
            row = c * _CHUNK + n
            for g in range(8):
                obuf[row, pl.ds(g * 16, 16)] = accs[g] * (1.0 / _DEG)

    start(0, buf0, sem0)
    start(1, buf1, sem1)

    def body2(j, carry):
        c = 2 * j
        wait(buf0, sem0)
        compute(c, buf0)

        @pl.when(c + 2 < _NCHUNKS)
        def _():
            start(c + 2, buf0, sem0)

        wait(buf1, sem1)
        compute(c + 1, buf1)

        @pl.when(c + 3 < _NCHUNKS)
        def _():
            start(c + 3, buf1, sem1)

        return carry

    lax.fori_loop(0, _NCHUNKS // 2, body2, 0)
    pltpu.sync_copy(obuf, out_hbm.at[wid])


_sc_mean = functools.partial(
    pl.kernel,
    out_type=jax.ShapeDtypeStruct((_NW, _PER_W, _D), jnp.float32),
    mesh=plsc.VectorSubcoreMesh(
        core_axis_name="c", subcore_axis_name="s", num_cores=_NC, num_subcores=_NS
    ),
    scratch_types=[
        pltpu.VMEM((_CHUNK, _DEG, _D), jnp.float32),
        pltpu.VMEM((_CHUNK, _DEG, _D), jnp.float32),
        pltpu.VMEM((_PER_W, _D), jnp.float32),
        pltpu.SemaphoreType.DMA,
        pltpu.SemaphoreType.DMA,
    ],
)(_sc_mean_body)


def _tc_tail_body(src_ref, neigh_ref, wa_ref, ws_ref, out_ref):
    mean = jnp.mean(neigh_ref[...], axis=1)
    h = jnp.dot(mean, wa_ref[...], preferred_element_type=jnp.float32)
    h += jnp.dot(src_ref[...], ws_ref[...], preferred_element_type=jnp.float32)
    out_ref[...] = jnp.maximum(h, 0.0)


def _tc_head_body(mean_ref, src_ref, wa_ref, ws_ref, out_ref):
    h = jnp.dot(mean_ref[...], wa_ref[...], preferred_element_type=jnp.float32)
    h += jnp.dot(src_ref[...], ws_ref[...], preferred_element_type=jnp.float32)
    out_ref[...] = jnp.maximum(h, 0.0)


def kernel(src_node_features, neighbor_node_features, W_agg, W_self):
    n_tail = _N - _A
    off = _A // _BN

    sc_mean = _sc_mean(neighbor_node_features).reshape(_A, _D)

    out_tail = pl.pallas_call(
        _tc_tail_body,
        grid=(n_tail // _BN,),
        in_specs=[
            pl.BlockSpec((_BN, _D), lambda i: (i + off, 0)),
            pl.BlockSpec((_BN, _DEG, _D), lambda i: (i + off, 0, 0)),
            pl.BlockSpec((_D, _D), lambda i: (0, 0)),
            pl.BlockSpec((_D, _D), lambda i: (0, 0)),
        ],
        out_specs=pl.BlockSpec((_BN, _D), lambda i: (i, 0)),
        out_shape=jax.ShapeDtypeStruct((n_tail, _D), jnp.float32),
    )(src_node_features, neighbor_node_features, W_agg, W_self)

    out_head = pl.pallas_call(
        _tc_head_body,
        grid=(_A // _BN,),
        in_specs=[
            pl.BlockSpec((_BN, _D), lambda i: (i, 0)),
            pl.BlockSpec((_BN, _D), lambda i: (i, 0)),
            pl.BlockSpec((_D, _D), lambda i: (0, 0)),
            pl.BlockSpec((_D, _D), lambda i: (0, 0)),
        ],
        out_specs=pl.BlockSpec((_BN, _D), lambda i: (i, 0)),
        out_shape=jax.ShapeDtypeStruct((_A, _D), jnp.float32),
    )(sc_mean, src_node_features, W_agg, W_self)

    return jnp.concatenate([out_head, out_tail], axis=0)


# speedup vs baseline: 1.5197x; 1.5197x over previous
import jax
import jax.numpy as jnp
from jax.experimental import pallas as pl

_BN = 1000


def _body(src_ref, neigh_ref, wa_ref, ws_ref, out_ref):
    mean = jnp.mean(neigh_ref[...], axis=1)
    h = jnp.dot(mean, wa_ref[...], preferred_element_type=jnp.float32)
    h += jnp.dot(src_ref[...], ws_ref[...], preferred_element_type=jnp.float32)
    out_ref[...] = jnp.maximum(h, 0.0)


def kernel(src_node_features, neighbor_node_features, W_agg, W_self):
    n, deg, d_in = neighbor_node_features.shape
    d_hid = W_agg.shape[1]
    grid = (n // _BN,)
    return pl.pallas_call(
        _body,
        grid=grid,
        in_specs=[
            pl.BlockSpec((_BN, d_in), lambda i: (i, 0)),
            pl.BlockSpec((_BN, deg, d_in), lambda i: (i, 0, 0)),
            pl.BlockSpec((d_in, d_hid), lambda i: (0, 0)),
            pl.BlockSpec((d_in, d_hid), lambda i: (0, 0)),
        ],
        out_specs=pl.BlockSpec((_BN, d_hid), lambda i: (i, 0)),
        out_shape=jax.ShapeDtypeStruct((n, d_hid), jnp.float32),
    )(src_node_features, neighbor_node_features, W_agg, W_self)
